# baseline (speedup 1.0000x reference)
import functools

import jax
import jax.numpy as jnp
from jax import lax
from jax.experimental import pallas as pl
from jax.experimental.pallas import tpu as pltpu

V, D, H, FF, L, E, TOPK = 32000, 768, 12, 2048, 2, 8, 2
B, S = 1, 2048
DH = D // H

_EPS = 1e-5


def _ln_rows(y, g, b):
    m = jnp.mean(y, axis=-1, keepdims=True)
    v = jnp.mean((y - m) ** 2, axis=-1, keepdims=True)
    return (y - m) * lax.rsqrt(v + _EPS) * g + b


_GROWS = 8


def _embed_kernel(ids_ref, *refs):
    del ids_ref
    out_ref = refs[_GROWS]
    for j in range(_GROWS):
        out_ref[j, :] = refs[j][0, 0, :]


def _embed(ids, emb):
    emb3 = emb.reshape(V, 1, D)
    grid_spec = pltpu.PrefetchScalarGridSpec(
        num_scalar_prefetch=1,
        grid=(S // _GROWS,),
        in_specs=[
            pl.BlockSpec((1, 1, D), functools.partial(
                lambda j, i, ids: (ids[i * _GROWS + j], 0, 0), j))
            for j in range(_GROWS)
        ],
        out_specs=pl.BlockSpec((_GROWS, D), lambda i, ids: (i, 0)),
    )
    return pl.pallas_call(
        _embed_kernel,
        grid_spec=grid_spec,
        out_shape=jax.ShapeDtypeStruct((S, D), jnp.float32),
    )(ids, *([emb3] * _GROWS))


def _qkv_kernel(x_ref, w_ref, b_ref, o_ref):
    y = lax.dot_general(x_ref[...], w_ref[...], (((1,), (1,)), ((), ())),
                        preferred_element_type=jnp.float32) + b_ref[0]
    o_ref[...] = y[None]


def _qkv_proj(x, Wqkv, bqkv):
    BM = 512
    return pl.pallas_call(
        _qkv_kernel,
        grid=(S // BM, 3 * H),
        in_specs=[
            pl.BlockSpec((BM, D), lambda i, c: (i, 0)),
            pl.BlockSpec((DH, D), lambda i, c: (c, 0)),
            pl.BlockSpec((1, 1, DH), lambda i, c: (c, 0, 0)),
        ],
        out_specs=pl.BlockSpec((1, BM, DH), lambda i, c: (c, i, 0)),
        out_shape=jax.ShapeDtypeStruct((3 * H, S, DH), jnp.float32),
    )(x, Wqkv, bqkv.reshape(3 * H, 1, DH))


def _attn_kernel(q_ref, k_ref, v_ref, o_ref):
    q = q_ref[0]
    k = k_ref[0]
    s = lax.dot_general(q, k, (((1,), (1,)), ((), ())),
                        preferred_element_type=jnp.float32)
    s = s * (1.0 / (DH ** 0.5))
    m = jnp.max(s, axis=-1, keepdims=True)
    p = jnp.exp(s - m)
    denom = jnp.sum(p, axis=-1, keepdims=True)
    o = jnp.dot(p, v_ref[0], preferred_element_type=jnp.float32)
    o_ref[...] = (o / denom)[None]


def _attention(qkv):
    BM = 512
    return pl.pallas_call(
        _attn_kernel,
        grid=(H, S // BM),
        in_specs=[
            pl.BlockSpec((1, BM, DH), lambda h, i: (h, i, 0)),
            pl.BlockSpec((1, S, DH), lambda h, i: (H + h, 0, 0)),
            pl.BlockSpec((1, S, DH), lambda h, i: (2 * H + h, 0, 0)),
        ],
        out_specs=pl.BlockSpec((1, BM, DH), lambda h, i: (h, i, 0)),
        out_shape=jax.ShapeDtypeStruct((H, S, DH), jnp.float32),
    )(qkv, qkv, qkv)


def _oproj_ln_kernel(o_ref, w_ref, b_ref, r_ref, g_ref, bb_ref, out_ref):
    bm = r_ref.shape[0]
    y = b_ref[...] + r_ref[...]
    for h in range(H):
        y = y + lax.dot_general(
            o_ref[h], w_ref[:, h, :], (((1,), (1,)), ((), ())),
            preferred_element_type=jnp.float32)
    out_ref[...] = _ln_rows(y, g_ref[...], bb_ref[...])


def _oproj_ln(o, Wo, bo, resid, g, b):
    BM = 512
    return pl.pallas_call(
        _oproj_ln_kernel,
        grid=(S // BM,),
        in_specs=[
            pl.BlockSpec((H, BM, DH), lambda i: (0, i, 0)),
            pl.BlockSpec((D, H, DH), lambda i: (0, 0, 0)),
            pl.BlockSpec((1, D), lambda i: (0, 0)),
            pl.BlockSpec((BM, D), lambda i: (i, 0)),
            pl.BlockSpec((1, D), lambda i: (0, 0)),
            pl.BlockSpec((1, D), lambda i: (0, 0)),
        ],
        out_specs=pl.BlockSpec((BM, D), lambda i: (i, 0)),
        out_shape=jax.ShapeDtypeStruct((S, D), jnp.float32),
    )(o, Wo.reshape(D, H, DH), bo.reshape(1, D), resid,
      g.reshape(1, D), b.reshape(1, D))


def _gate_kernel(x_ref, gw_ref, gb_ref, w_ref):
    gs = lax.dot_general(x_ref[...], gw_ref[...], (((1,), (1,)), ((), ())),
                         preferred_element_type=jnp.float32) + gb_ref[...]
    n = gs.shape[0]
    ii = lax.broadcasted_iota(jnp.int32, (n, E), 1)
    a1 = jnp.argmax(gs, axis=-1)[:, None]
    m1 = jnp.max(gs, axis=-1, keepdims=True)
    gs2 = jnp.where(ii == a1, -jnp.inf, gs)
    a2 = jnp.argmax(gs2, axis=-1)[:, None]
    m2 = jnp.max(gs2, axis=-1, keepdims=True)
    p1 = 1.0 / (1.0 + jnp.exp(m2 - m1))
    p2 = 1.0 - p1
    w_ref[...] = jnp.where(ii == a1, p1, 0.0) + jnp.where(ii == a2, p2, 0.0)


def _gate(x, gW, gb):
    BM = 1024
    return pl.pallas_call(
        _gate_kernel,
        grid=(S // BM,),
        in_specs=[
            pl.BlockSpec((BM, D), lambda i: (i, 0)),
            pl.BlockSpec((E, D), lambda i: (0, 0)),
            pl.BlockSpec((1, E), lambda i: (0, 0)),
        ],
        out_specs=pl.BlockSpec((BM, E), lambda i: (i, 0)),
        out_shape=jax.ShapeDtypeStruct((S, E), jnp.float32),
    )(x, gW, gb.reshape(1, E))


def _moe_kernel(x_ref, w1_ref, b1_ref, w2_ref, b2_ref, wt_ref,
                g_ref, bb_ref, out_ref, acc_ref):
    e = pl.program_id(0)
    i = pl.program_id(1)
    bm = x_ref.shape[0]

    @pl.when(e == 0)
    def _():
        acc_ref[pl.ds(i * bm, bm), :] = jnp.zeros((bm, D), jnp.float32)

    ii = lax.broadcasted_iota(jnp.int32, (bm, E), 1)
    wcol = jnp.sum(wt_ref[...] * (ii == e).astype(jnp.float32),
                   axis=1, keepdims=True)
    h = lax.dot_general(x_ref[...], w1_ref[0], (((1,), (1,)), ((), ())),
                        preferred_element_type=jnp.float32) + b1_ref[0]
    h = jnp.maximum(h, 0.0)
    eo = lax.dot_general(h, w2_ref[0], (((1,), (1,)), ((), ())),
                         preferred_element_type=jnp.float32) + b2_ref[0]
    acc_ref[pl.ds(i * bm, bm), :] += wcol * eo

    @pl.when(e == E - 1)
    def _():
        y = x_ref[...] + acc_ref[pl.ds(i * bm, bm), :]
        out_ref[...] = _ln_rows(y, g_ref[...], bb_ref[...])


def _moe_ln(x, W1, b1, W2, b2, wt, g, b):
    BM = 1024
    return pl.pallas_call(
        _moe_kernel,
        grid=(E, S // BM),
        in_specs=[
            pl.BlockSpec((BM, D), lambda e, i: (i, 0)),
            pl.BlockSpec((1, FF, D), lambda e, i: (e, 0, 0)),
            pl.BlockSpec((1, 1, FF), lambda e, i: (e, 0, 0)),
            pl.BlockSpec((1, D, FF), lambda e, i: (e, 0, 0)),
            pl.BlockSpec((1, 1, D), lambda e, i: (e, 0, 0)),
            pl.BlockSpec((BM, E), lambda e, i: (i, 0)),
            pl.BlockSpec((1, D), lambda e, i: (0, 0)),
            pl.BlockSpec((1, D), lambda e, i: (0, 0)),
        ],
        out_specs=pl.BlockSpec((BM, D), lambda e, i: (i, 0)),
        out_shape=jax.ShapeDtypeStruct((S, D), jnp.float32),
        scratch_shapes=[pltpu.VMEM((S, D), jnp.float32)],
    )(x, W1, b1.reshape(E, 1, FF), W2, b2.reshape(E, 1, D), wt,
      g.reshape(1, D), b.reshape(1, D))


def _head_kernel(x_ref, g_ref, b_ref, w_ref, hb_ref, o_ref):
    xb = _ln_rows(x_ref[...], g_ref[...], b_ref[...])
    o_ref[...] = lax.dot_general(
        xb, w_ref[...], (((1,), (1,)), ((), ())),
        preferred_element_type=jnp.float32) + hb_ref[...]


def _head(x, lfg, lfb, hW, hb):
    BM, BN = 512, 1280
    return pl.pallas_call(
        _head_kernel,
        grid=(S // BM, V // BN),
        in_specs=[
            pl.BlockSpec((BM, D), lambda i, j: (i, 0)),
            pl.BlockSpec((1, D), lambda i, j: (0, 0)),
            pl.BlockSpec((1, D), lambda i, j: (0, 0)),
            pl.BlockSpec((BN, D), lambda i, j: (j, 0)),
            pl.BlockSpec((1, BN), lambda i, j: (0, j)),
        ],
        out_specs=pl.BlockSpec((BM, BN), lambda i, j: (i, j)),
        out_shape=jax.ShapeDtypeStruct((S, V), jnp.float32),
    )(x, lfg.reshape(1, D), lfb.reshape(1, D), hW, hb.reshape(1, V))


def kernel(input_ids, emb, Wqkv, bqkv, Wo, bo, gW, gb, W1, b1, W2, b2,
           n1g, n1b, n2g, n2b, lfg, lfb, hW, hb):
    ids = input_ids.reshape(S).astype(jnp.int32)
    x = _embed(ids, emb)
    for l in range(L):
        qkv = _qkv_proj(x, Wqkv[l], bqkv[l])
        o = _attention(qkv)
        x = _oproj_ln(o, Wo[l], bo[l], x, n1g[l], n1b[l])
        wt = _gate(x, gW[l], gb[l])
        x = _moe_ln(x, W1[l], b1[l], W2[l], b2[l], wt, n2g[l], n2b[l])
    out = _head(x, lfg, lfb, hW, hb)
    return out.reshape(B, S, V)
